# baseline (speedup 1.0000x reference)
import jax
import jax.numpy as jnp
from jax.experimental import pallas as pl
from jax.experimental.pallas import tpu as pltpu

_ROWS_PER_STEP = 2048
_STREAMS = 8


def _gather_kernel(*refs, rows, streams):
    idx_refs = refs[:streams]
    tab_ref = refs[streams]
    out_ref = refs[streams + 1]
    for k in range(rows // streams):
        for j in range(streams):
            v = idx_refs[j][0, 0, k]
            out_ref[k * streams + j, :] = tab_ref[v, 0, :]


def kernel(table, x):
    V, D = table.shape
    B, F = x.shape
    N = B * F

    R = min(_ROWS_PER_STEP, N)
    n_steps = pl.cdiv(N, R)
    N_pad = n_steps * R

    flat_idx = x.reshape(-1).astype(jnp.int32)
    if N_pad != N:
        flat_idx = jnp.pad(flat_idx, (0, N_pad - N))

    J = _STREAMS if R % _STREAMS == 0 else 1
    K = R // J
    idx_kj = flat_idx.reshape(n_steps, K, J)
    idx_streams = [idx_kj[:, :, j].reshape(n_steps, 1, K) for j in range(J)]

    tab3 = table.reshape(V, 1, D)

    out = pl.pallas_call(
        lambda *refs: _gather_kernel(*refs, rows=R, streams=J),
        out_shape=jax.ShapeDtypeStruct((N_pad, D), table.dtype),
        grid=(n_steps,),
        in_specs=[
            pl.BlockSpec((1, 1, K), lambda i: (i, 0, 0),
                         memory_space=pltpu.SMEM)
            for _ in range(J)
        ] + [
            pl.BlockSpec((V, 1, D), lambda i: (0, 0, 0)),
        ],
        out_specs=pl.BlockSpec((R, D), lambda i: (i, 0)),
        compiler_params=pltpu.CompilerParams(
            dimension_semantics=("arbitrary",),
            vmem_limit_bytes=48 * 1024 * 1024),
        cost_estimate=pl.CostEstimate(
            flops=0,
            transcendentals=0,
            bytes_accessed=N_pad * 4 + V * D * 4 + N_pad * D * 4),
    )(*idx_streams, tab3)

    if N_pad != N:
        out = out[:N]
    return out.reshape(B, F, D)


# speedup vs baseline: 1.3011x; 1.3011x over previous
import jax
import jax.numpy as jnp
from jax.experimental import pallas as pl
from jax.experimental.pallas import tpu as pltpu

_ROWS_PER_STEP = 2048
_STREAMS = 8


def _gather_kernel(*refs, rows, streams):
    idx_refs = refs[:streams]
    tab_ref = refs[streams]
    out_ref = refs[streams + 1]
    for k in range(rows // streams):
        for j in range(streams):
            v = idx_refs[j][0, 0, k]
            out_ref[k * streams + j, :] = tab_ref[v, 0, :]


def kernel(table, x):
    V, D = table.shape
    B, F = x.shape
    N = B * F

    R = min(_ROWS_PER_STEP, N)
    n_steps = pl.cdiv(N, R)
    N_pad = n_steps * R

    flat_idx = x.reshape(-1).astype(jnp.int32)
    if N_pad != N:
        flat_idx = jnp.pad(flat_idx, (0, N_pad - N))

    J = _STREAMS if R % _STREAMS == 0 else 1
    K = R // J
    Kp = K + 1
    idx_kj = flat_idx.reshape(n_steps, K, J)
    idx_streams = [
        jnp.pad(idx_kj[:, :, j], ((0, 0), (0, Kp - K))).reshape(n_steps, 1, Kp)
        for j in range(J)
    ]

    tab3 = table.reshape(V, 1, D)

    out = pl.pallas_call(
        lambda *refs: _gather_kernel(*refs, rows=R, streams=J),
        out_shape=jax.ShapeDtypeStruct((N_pad, D), table.dtype),
        grid=(n_steps,),
        in_specs=[
            pl.BlockSpec((1, 1, Kp), lambda i: (i, 0, 0),
                         memory_space=pltpu.SMEM)
            for _ in range(J)
        ] + [
            pl.BlockSpec((V, 1, D), lambda i: (0, 0, 0)),
        ],
        out_specs=pl.BlockSpec((R, D), lambda i: (i, 0)),
        compiler_params=pltpu.CompilerParams(
            dimension_semantics=("arbitrary",),
            vmem_limit_bytes=48 * 1024 * 1024),
        cost_estimate=pl.CostEstimate(
            flops=0,
            transcendentals=0,
            bytes_accessed=N_pad * 4 + V * D * 4 + N_pad * D * 4),
    )(*idx_streams, tab3)

    if N_pad != N:
        out = out[:N]
    return out.reshape(B, F, D)
